# Initial kernel scaffold; baseline (speedup 1.0000x reference)
import jax
import jax.numpy as jnp
from jax.experimental import pallas as pl


def kernel(x, expert_p, expert_idxs, W1, W2):
    raise NotImplementedError("write your pallas kernel here")


# speedup vs baseline: 4.4455x; 4.4455x over previous
import functools

import jax
import jax.numpy as jnp
from jax.experimental import pallas as pl
from jax.experimental.pallas import tpu as pltpu

BT = 128


def _gemm_body(eb_ref, valid_ref, xs_ref, w1_ref, w2_ref, ps_ref, ys_ref, *, h):
    g = pl.program_id(0)

    @pl.when(valid_ref[g] == 1)
    def _():
        x_blk = xs_ref[...]
        he = jnp.dot(x_blk, w1_ref[0], preferred_element_type=jnp.float32)
        h_part = he[:, :h]
        gates = he[:, h:]
        act = h_part * (gates * jax.lax.logistic(gates))
        y = jnp.dot(act, w2_ref[0], preferred_element_type=jnp.float32)
        ys_ref[...] = y * ps_ref[0, 0, :][:, None]


def _grouped_gemm(xs, w1, w2, ps, eb, valid, *, G, d, h):
    P = G * BT
    grid_spec = pltpu.PrefetchScalarGridSpec(
        num_scalar_prefetch=2,
        grid=(G,),
        in_specs=[
            pl.BlockSpec((BT, d), lambda g, eb, vb: (g, 0)),
            pl.BlockSpec((1, d, 2 * h), lambda g, eb, vb: (eb[g], 0, 0)),
            pl.BlockSpec((1, h, d), lambda g, eb, vb: (eb[g], 0, 0)),
            pl.BlockSpec((1, 1, BT), lambda g, eb, vb: (g, 0, 0)),
        ],
        out_specs=pl.BlockSpec((BT, d), lambda g, eb, vb: (g, 0)),
    )
    return pl.pallas_call(
        functools.partial(_gemm_body, h=h),
        grid_spec=grid_spec,
        out_shape=jax.ShapeDtypeStruct((P, d), jnp.float32),
    )(eb, valid, xs, w1, w2, ps.reshape(G, 1, BT))


def kernel(x, expert_p, expert_idxs, W1, W2):
    x_shape = x.shape
    d = x_shape[-1]
    xf = x.reshape(-1, d)
    T, k = expert_idxs.shape
    E = W1.shape[0]
    h = W2.shape[1]
    assert k == 1

    G = T // BT + E
    P = G * BT

    flat = expert_idxs.reshape(-1).astype(jnp.int32)
    order = jnp.argsort(flat)
    sorted_e = flat[order]
    counts = jnp.zeros((E,), jnp.int32).at[flat].add(1)
    blocks_per_e = (counts + BT - 1) // BT
    block_start = jnp.concatenate(
        [jnp.zeros((1,), jnp.int32), jnp.cumsum(blocks_per_e)[:-1]])
    tok_start = jnp.concatenate(
        [jnp.zeros((1,), jnp.int32), jnp.cumsum(counts)[:-1]])
    num_active = jnp.sum(blocks_per_e)

    i = jnp.arange(T, dtype=jnp.int32)
    dest_sorted = block_start[sorted_e] * BT + (i - tok_start[sorted_e])

    g = jnp.arange(G, dtype=jnp.int32)
    eb_raw = jnp.searchsorted(block_start, g, side="right").astype(jnp.int32) - 1
    eb_last = jnp.take(eb_raw, num_active - 1)
    valid = (g < num_active).astype(jnp.int32)
    eb = jnp.where(valid == 1, eb_raw, eb_last)

    src = jnp.zeros((P,), jnp.int32).at[dest_sorted].set(order.astype(jnp.int32))
    ps = jnp.zeros((P,), jnp.float32).at[dest_sorted].set(
        expert_p.reshape(-1)[order])
    xs = xf[src]

    ys = _grouped_gemm(xs, W1, W2, ps, eb, valid, G=G, d=d, h=h)

    dest_tok = jnp.zeros((T,), jnp.int32).at[order].set(dest_sorted.astype(jnp.int32))
    out = ys[dest_tok]
    return out.reshape(*x_shape[:-1], d)


# speedup vs baseline: 6.3049x; 1.4183x over previous
import functools

import jax
import jax.numpy as jnp
from jax import lax
from jax.experimental import pallas as pl
from jax.experimental.pallas import tpu as pltpu
from jax.experimental.pallas import tpu_sc as plsc

BT = 128
NC = 2
NS = 16
CH = 32
L = 16


def _dispatch_body(T, E, d, G, idx_hbm, x_hbm, xs_hbm, dest_hbm, eb_hbm,
                   valid_hbm, idxv, cntv, call, destv, dchunks, ebv, validv,
                   xv, shared, cnt_s, cntg_s, prev_s, bs_s, base_s):
    c = lax.axis_index("c")
    s = lax.axis_index("s")
    tps = T // NS
    nv = tps // L
    zeros16 = jnp.zeros((L,), jnp.int32)
    lane = lax.iota(jnp.int32, L)

    pltpu.sync_copy(idx_hbm.at[pl.ds(s, 1)], idxv)
    for e in range(E):
        cnt_s[e] = 0
    for v in range(nv):
        vec = idxv[0, pl.ds(v * L, L)]
        for l in range(L):
            e = vec[l]
            cnt_s[e] = cnt_s[e] + 1

    for q in range(E // L):
        acc = zeros16
        for l in range(L):
            acc = jnp.where(lane == l, cnt_s[q * L + l], acc)
        cntv[pl.ds(q * L, L)] = acc
    pltpu.sync_copy(cntv, shared.at[pl.ds(s * E, E)])
    plsc.subcore_barrier()
    pltpu.sync_copy(shared, call)

    for q in range(E // L):
        acc = zeros16
        pre = zeros16
        for sp in range(NS):
            v = call[pl.ds(sp * E + q * L, L)]
            acc = acc + v
            pre = pre + v * jnp.where(sp < s, 1, 0)
        for l in range(L):
            cntg_s[q * L + l] = acc[l]
            prev_s[q * L + l] = pre[l]

    cblk = 0
    for e in range(E):
        bs_s[e] = cblk
        base_s[e] = cblk * BT + prev_s[e]
        cblk = cblk + (cntg_s[e] + BT - 1) // BT
    na = cblk

    for e in range(E):
        cnt_s[e] = 0
    for v in range(nv):
        vec = idxv[0, pl.ds(v * L, L)]
        dst = zeros16
        for l in range(L):
            e = vec[l]
            r = cnt_s[e]
            cnt_s[e] = r + 1
            dst = jnp.where(lane == l, base_s[e] + r, dst)
        destv[pl.ds(v * L, L)] = dst
        dchunks[v // (CH // L)][pl.ds((v % (CH // L)) * L, L)] = dst
    pltpu.sync_copy(destv, dest_hbm.at[pl.ds(s * tps, tps)])

    for q in range(G // L):
        gv = lane + q * L
        ggv = jnp.minimum(gv, na - 1)
        acc = zeros16
        for e in range(E):
            acc = acc + jnp.where(bs_s[e] <= ggv, 1, 0)
        ebv[pl.ds(q * L, L)] = acc - 1
        validv[pl.ds(q * L, L)] = jnp.where(gv < na, 1, 0)
    pltpu.sync_copy(ebv, eb_hbm)
    pltpu.sync_copy(validv, valid_hbm)

    for j in range(tps // CH):
        pltpu.sync_copy(x_hbm.at[pl.ds(s * tps + j * CH, CH)], xv)
        pltpu.sync_copy(xv, xs_hbm.at[dchunks[j]])


def _dispatch(idx2, xf, *, T, E, d, G):
    P = G * BT
    tps = T // NS
    mesh = plsc.VectorSubcoreMesh(core_axis_name="c", subcore_axis_name="s")
    body = functools.partial(_dispatch_body, T, E, d, G)
    return pl.kernel(
        body,
        out_type=[
            jax.ShapeDtypeStruct((P, d), jnp.float32),
            jax.ShapeDtypeStruct((T,), jnp.int32),
            jax.ShapeDtypeStruct((G,), jnp.int32),
            jax.ShapeDtypeStruct((G,), jnp.int32),
        ],
        mesh=mesh,
        scratch_types=[
            pltpu.VMEM((1, tps), jnp.int32),
            pltpu.VMEM((E,), jnp.int32),
            pltpu.VMEM((NS * E,), jnp.int32),
            pltpu.VMEM((tps,), jnp.int32),
            [pltpu.VMEM((CH,), jnp.int32)] * 4,
            pltpu.VMEM((G,), jnp.int32),
            pltpu.VMEM((G,), jnp.int32),
            pltpu.VMEM((CH, d), jnp.float32),
            pltpu.VMEM_SHARED((NS * E,), jnp.int32),
            pltpu.SMEM((E,), jnp.int32),
            pltpu.SMEM((E,), jnp.int32),
            pltpu.SMEM((E,), jnp.int32),
            pltpu.SMEM((E,), jnp.int32),
            pltpu.SMEM((E,), jnp.int32),
        ],
    )(idx2, xf)


def _gemm_body(eb_ref, valid_ref, xs_ref, w1_ref, w2_ref, ys_ref, *, h):
    g = pl.program_id(0)

    @pl.when(valid_ref[g] == 1)
    def _():
        x_blk = xs_ref[...]
        he = jnp.dot(x_blk, w1_ref[0], preferred_element_type=jnp.float32)
        h_part = he[:, :h]
        gates = he[:, h:]
        act = h_part * (gates * jax.lax.logistic(gates))
        ys_ref[...] = jnp.dot(act, w2_ref[0],
                              preferred_element_type=jnp.float32)


def _grouped_gemm(xs, w1, w2, eb, valid, *, G, d, h):
    P = G * BT
    grid_spec = pltpu.PrefetchScalarGridSpec(
        num_scalar_prefetch=2,
        grid=(G,),
        in_specs=[
            pl.BlockSpec((BT, d), lambda g, eb, vb: (g, 0)),
            pl.BlockSpec((1, d, 2 * h), lambda g, eb, vb: (eb[g], 0, 0)),
            pl.BlockSpec((1, h, d), lambda g, eb, vb: (eb[g], 0, 0)),
        ],
        out_specs=pl.BlockSpec((BT, d), lambda g, eb, vb: (g, 0)),
    )
    return pl.pallas_call(
        functools.partial(_gemm_body, h=h),
        grid_spec=grid_spec,
        out_shape=jax.ShapeDtypeStruct((P, d), jnp.float32),
    )(eb, valid, xs, w1, w2)


def _combine_body(T, d, ys_hbm, dest_hbm, p_hbm, out_hbm, dchunk, pv, yv, sem):
    c = lax.axis_index("c")
    s = lax.axis_index("s")
    w = s * NC + c
    tpw = T // (NC * NS)

    pltpu.sync_copy(p_hbm.at[pl.ds(w * tpw, tpw)], pv)
    for j in range(tpw // CH):
        pltpu.sync_copy(dest_hbm.at[pl.ds(w * tpw + j * CH, CH)], dchunk)
        pltpu.async_copy(ys_hbm.at[dchunk], yv, sem).wait()
        for i in range(CH):
            t = j * CH + i
            pvec = pv[pl.ds((t // L) * L, L)]
            pi = pvec[t % L]

            def q_body(q, carry):
                yv[i, pl.ds(q * L, L)] = yv[i, pl.ds(q * L, L)] * pi
                return carry

            lax.fori_loop(0, d // L, q_body, 0)
        pltpu.sync_copy(yv, out_hbm.at[pl.ds(w * tpw + j * CH, CH)])


def _combine(ys, dest, p_flat, *, T, d):
    mesh = plsc.VectorSubcoreMesh(core_axis_name="c", subcore_axis_name="s")
    tpw = T // (NC * NS)
    body = functools.partial(_combine_body, T, d)
    return pl.kernel(
        body,
        out_type=jax.ShapeDtypeStruct((T, d), jnp.float32),
        mesh=mesh,
        scratch_types=[
            pltpu.VMEM((CH,), jnp.int32),
            pltpu.VMEM((tpw,), jnp.float32),
            pltpu.VMEM((CH, d), jnp.float32),
            pltpu.SemaphoreType.DMA,
        ],
    )(ys, dest, p_flat)


def kernel(x, expert_p, expert_idxs, W1, W2):
    x_shape = x.shape
    d = x_shape[-1]
    xf = x.reshape(-1, d)
    T, k = expert_idxs.shape
    E = W1.shape[0]
    h = W2.shape[1]
    assert k == 1

    G = T // BT + E

    idx2 = expert_idxs.reshape(NS, T // NS).astype(jnp.int32)
    xs, dest, eb, valid = _dispatch(idx2, xf, T=T, E=E, d=d, G=G)
    ys = _grouped_gemm(xs, W1, W2, eb, valid, G=G, d=d, h=h)
    out = _combine(ys, dest, expert_p.reshape(-1), T=T, d=d)
    return out.reshape(*x_shape[:-1], d)
